# Initial kernel scaffold; baseline (speedup 1.0000x reference)
import jax
import jax.numpy as jnp
from jax.experimental import pallas as pl


def kernel(X):
    raise NotImplementedError("write your pallas kernel here")


# speedup vs baseline: 59.2928x; 59.2928x over previous
import functools

import jax
import jax.numpy as jnp
from jax import lax
from jax.experimental import pallas as pl
from jax.experimental.pallas import tpu as pltpu
from jax.experimental.pallas import tpu_sc as plsc

B = 128
N = 32768
K = 64
L = 16
SHIFT = 21
NBINS = 1 << (32 - SHIFT)
NVECS = N // L
MASK31 = 0x7FFFFFFF
SENTINEL = -0x80000000


def _okey(x):
    b = lax.bitcast_convert_type(x, jnp.int32)
    return b ^ ((b >> 31) & MASK31)


def _masked_topk_kernel(x_hbm, out_hbm, row_v, hist_v, uc_v, ic_v):
    info = plsc.get_sparse_core_info()
    nc, ns = info.num_cores, info.num_subcores
    wid = lax.axis_index("s") * nc + lax.axis_index("c")
    rows_per = B // (nc * ns)

    iota = jnp.arange(L, dtype=jnp.int32)
    ones = jnp.ones((L,), jnp.int32)
    zeros = jnp.zeros((L,), jnp.int32)

    def do_row(r, _):
        row = wid * rows_per + r
        pltpu.sync_copy(x_hbm.at[row], row_v)

        def zero_body(i, c):
            hist_v[pl.ds(i * L, L)] = zeros
            return c

        lax.fori_loop(0, NBINS // L, zero_body, 0)

        def hist_body(i, c):
            x = row_v[pl.ds(i * L, L)]
            u = _okey(x)
            binv = (u >> SHIFT) + (NBINS // 2)
            plsc.addupdate_scatter(hist_v, [binv], ones)
            return c

        lax.fori_loop(0, NVECS, hist_body, 0)

        def scan_body(j, carry):
            total, bchunk, above = carry
            c = (NBINS // L - 1) - j
            v = hist_v[pl.ds(c * L, L)]
            csum = jnp.sum(v)
            hit = jnp.logical_and(total + csum >= K, bchunk < 0)
            bchunk = jnp.where(hit, c, bchunk)
            above = jnp.where(hit, total, above)
            return total + csum, bchunk, above

        _, bchunk, above = lax.fori_loop(
            0, NBINS // L, scan_body, (jnp.int32(0), jnp.int32(-1), jnp.int32(0))
        )

        v = hist_v[pl.ds(bchunk * L, L)]
        rv = lax.rev(v, (0,))
        cs = plsc.cumsum(rv)
        ok = (above + cs) >= K
        big = jnp.int32(0x7FFFFFF0)
        cs_at_p = jnp.min(jnp.where(ok, cs, big))
        prev = jnp.max(jnp.where(ok, 0, cs))
        cnt_b = cs_at_p - prev
        p = jnp.sum(jnp.where(ok, 0, 1))
        b = bchunk * L + (L - 1 - p)
        count_ge_b = above + cs_at_p
        need = K - (count_ge_b - cnt_b)

        def collect_body(i, ptr):
            x = row_v[pl.ds(i * L, L)]
            u = _okey(x)
            eq = ((u >> SHIFT) + (NBINS // 2)) == b
            idxv = iota + i * L
            plsc.store_compressed(uc_v.at[pl.ds(ptr, L)], u, mask=eq)
            plsc.store_compressed(ic_v.at[pl.ds(ptr, L)], idxv, mask=eq)
            cnt = jnp.max(plsc.all_reduce_population_count(eq))
            return ptr + cnt

        cand = lax.fori_loop(0, NVECS, collect_body, jnp.int32(0))
        uc_v[pl.ds(cand, L)] = jnp.full((L,), SENTINEL, jnp.int32)
        ic_v[pl.ds(cand, L)] = jnp.full((L,), -1, jnp.int32)
        nvec = (cand + L - 1) // L

        lo0 = (b - NBINS // 2) << SHIFT

        def count_ge_key(mid):
            def cb(i, acc):
                uv = uc_v[pl.ds(i * L, L)]
                return acc + jnp.sum(jnp.where(uv >= mid, 1, 0))

            return lax.fori_loop(0, nvec, cb, jnp.int32(0))

        def vsearch(j, lohi):
            lo, hi = lohi
            mid = lo + (hi - lo) // 2
            geq = count_ge_key(mid) >= need
            return jnp.where(geq, mid, lo), jnp.where(geq, hi, mid)

        t_lo, _ = lax.fori_loop(
            0, SHIFT, vsearch, (lo0, lo0 + jnp.int32(1 << SHIFT))
        )
        tkey = t_lo

        def cgt_body(i, acc):
            uv = uc_v[pl.ds(i * L, L)]
            return acc + jnp.sum(jnp.where(uv > tkey, 1, 0))

        cnt_gt = lax.fori_loop(0, nvec, cgt_body, jnp.int32(0))
        need_eq = need - cnt_gt

        def count_ge_idx(mid):
            def cb(i, acc):
                uv = uc_v[pl.ds(i * L, L)]
                iv = ic_v[pl.ds(i * L, L)]
                m = jnp.logical_and(uv == tkey, iv >= mid)
                return acc + jnp.sum(jnp.where(m, 1, 0))

            return lax.fori_loop(0, nvec, cb, jnp.int32(0))

        def isearch(j, lohi):
            lo, hi = lohi
            mid = lo + (hi - lo) // 2
            geq = count_ge_idx(mid) >= need_eq
            return jnp.where(geq, mid, lo), jnp.where(geq, hi, mid)

        icut, _ = lax.fori_loop(0, 15, isearch, (jnp.int32(0), jnp.int32(N)))

        def mask_body(i, c):
            sl = pl.ds(i * L, L)
            x = row_v[sl]
            u = _okey(x)
            idxv = iota + i * L
            keep = jnp.logical_or(
                u > tkey, jnp.logical_and(u == tkey, idxv >= icut)
            )
            row_v[sl] = jnp.where(keep, x, jnp.float32(0.0))
            return c

        lax.fori_loop(0, NVECS, mask_body, 0)
        pltpu.sync_copy(row_v, out_hbm.at[row])
        return 0

    lax.fori_loop(0, rows_per, do_row, 0)


@jax.jit
def kernel(X):
    mesh = plsc.VectorSubcoreMesh(core_axis_name="c", subcore_axis_name="s")
    f = functools.partial(
        pl.kernel,
        mesh=mesh,
        out_type=jax.ShapeDtypeStruct((B, N), jnp.float32),
        scratch_types=[
            pltpu.VMEM((N,), jnp.float32),
            pltpu.VMEM((NBINS,), jnp.int32),
            pltpu.VMEM((N + L,), jnp.int32),
            pltpu.VMEM((N + L,), jnp.int32),
        ],
        compiler_params=pltpu.CompilerParams(needs_layout_passes=False),
    )(_masked_topk_kernel)
    return f(X)


# speedup vs baseline: 73.4718x; 1.2391x over previous
import functools

import jax
import jax.numpy as jnp
from jax import lax
from jax.experimental import pallas as pl
from jax.experimental.pallas import tpu as pltpu
from jax.experimental.pallas import tpu_sc as plsc

B = 128
N = 32768
K = 64
L = 16
SHIFT = 21
NBINS = 1 << (32 - SHIFT)
HALF = NBINS // 2
NCHUNK = NBINS // L
NVECS = N // L
U = 8
MASK31 = 0x7FFFFFFF
SENTINEL = -0x80000000


def _okey(x):
    b = lax.bitcast_convert_type(x, jnp.int32)
    return b ^ ((b >> 31) & MASK31)


def _masked_topk_kernel(x_hbm, out_hbm, row_v, hist_v, uc_v, ic_v):
    info = plsc.get_sparse_core_info()
    nc, ns = info.num_cores, info.num_subcores
    wid = lax.axis_index("s") * nc + lax.axis_index("c")
    rows_per = B // (nc * ns)

    iota = jnp.arange(L, dtype=jnp.int32)
    ones = jnp.ones((L,), jnp.int32)
    zeros = jnp.zeros((L,), jnp.int32)

    def do_row(r, _):
        row = wid * rows_per + r
        pltpu.sync_copy(x_hbm.at[row], row_v)

        def zero_body(g, c):
            for j in range(U):
                hist_v[pl.ds((g * U + j) * L, L)] = zeros
            return c

        lax.fori_loop(0, NCHUNK // U, zero_body, 0)

        def hist_body(g, c):
            for j in range(U):
                x = row_v[pl.ds((g * U + j) * L, L)]
                binv = (_okey(x) >> SHIFT) + HALF
                plsc.addupdate_scatter(hist_v, [binv], ones)
            return c

        lax.fori_loop(0, NVECS // U, hist_body, 0)

        def scan_body(gj, carry):
            total, bchunk, above = carry
            for j in range(U):
                c = (NCHUNK - 1) - (gj * U + j)
                v = hist_v[pl.ds(c * L, L)]
                csum = jnp.sum(v)
                hit = jnp.logical_and(total + csum >= K, bchunk < 0)
                bchunk = jnp.where(hit, c, bchunk)
                above = jnp.where(hit, total, above)
                total = total + csum
            return total, bchunk, above

        _, bchunk, above = lax.fori_loop(
            0,
            NCHUNK // U,
            scan_body,
            (jnp.int32(0), jnp.int32(-1), jnp.int32(0)),
        )

        v = hist_v[pl.ds(bchunk * L, L)]
        rv = lax.rev(v, (0,))
        cs = plsc.cumsum(rv)
        ok = (above + cs) >= K
        big = jnp.int32(0x7FFFFFF0)
        cs_at_p = jnp.min(jnp.where(ok, cs, big))
        prev = jnp.max(jnp.where(ok, 0, cs))
        cnt_b = cs_at_p - prev
        p = jnp.sum(jnp.where(ok, 0, 1))
        b = bchunk * L + (L - 1 - p)
        count_ge_b = above + cs_at_p
        need = K - (count_ge_b - cnt_b)

        def collect_body(i, ptr):
            x = row_v[pl.ds(i * L, L)]
            u = _okey(x)
            eq = ((u >> SHIFT) + HALF) == b
            idxv = iota + i * L
            plsc.store_compressed(uc_v.at[pl.ds(ptr, L)], u, mask=eq)
            plsc.store_compressed(ic_v.at[pl.ds(ptr, L)], idxv, mask=eq)
            cnt = jnp.max(plsc.all_reduce_population_count(eq))
            return ptr + cnt

        cand = lax.fori_loop(0, NVECS, collect_body, jnp.int32(0))
        uc_v[pl.ds(cand, L)] = jnp.full((L,), SENTINEL, jnp.int32)
        ic_v[pl.ds(cand, L)] = jnp.full((L,), -1, jnp.int32)
        nvec = (cand + L - 1) // L

        lo0 = (b - HALF) << SHIFT

        def count_ge_key(mid):
            def cb(i, accv):
                uv = uc_v[pl.ds(i * L, L)]
                return accv + jnp.where(uv >= mid, ones, zeros)

            return jnp.sum(lax.fori_loop(0, nvec, cb, zeros))

        def vsearch(j, lohi):
            lo, hi = lohi
            mid = lo + (hi - lo) // 2
            geq = count_ge_key(mid) >= need
            return jnp.where(geq, mid, lo), jnp.where(geq, hi, mid)

        tkey, _ = lax.fori_loop(
            0, SHIFT, vsearch, (lo0, lo0 + jnp.int32(1 << SHIFT))
        )

        def cgt_body(i, accv):
            uv = uc_v[pl.ds(i * L, L)]
            return accv + jnp.where(uv > tkey, ones, zeros)

        cnt_gt = jnp.sum(lax.fori_loop(0, nvec, cgt_body, zeros))
        need_eq = need - cnt_gt

        def count_ge_idx(mid):
            def cb(i, accv):
                uv = uc_v[pl.ds(i * L, L)]
                iv = ic_v[pl.ds(i * L, L)]
                m = jnp.logical_and(uv == tkey, iv >= mid)
                return accv + jnp.where(m, ones, zeros)

            return jnp.sum(lax.fori_loop(0, nvec, cb, zeros))

        def isearch(j, lohi):
            lo, hi = lohi
            mid = lo + (hi - lo) // 2
            geq = count_ge_idx(mid) >= need_eq
            return jnp.where(geq, mid, lo), jnp.where(geq, hi, mid)

        icut, _ = lax.fori_loop(0, 15, isearch, (jnp.int32(0), jnp.int32(N)))

        def mask_body(g, c):
            for j in range(U):
                i = g * U + j
                sl = pl.ds(i * L, L)
                x = row_v[sl]
                u = _okey(x)
                idxv = iota + i * L
                keep = jnp.logical_or(
                    u > tkey, jnp.logical_and(u == tkey, idxv >= icut)
                )
                row_v[sl] = jnp.where(keep, x, jnp.float32(0.0))
            return c

        lax.fori_loop(0, NVECS // U, mask_body, 0)
        pltpu.sync_copy(row_v, out_hbm.at[row])
        return 0

    lax.fori_loop(0, rows_per, do_row, 0)


@jax.jit
def kernel(X):
    mesh = plsc.VectorSubcoreMesh(core_axis_name="c", subcore_axis_name="s")
    f = functools.partial(
        pl.kernel,
        mesh=mesh,
        out_type=jax.ShapeDtypeStruct((B, N), jnp.float32),
        scratch_types=[
            pltpu.VMEM((N,), jnp.float32),
            pltpu.VMEM((NBINS,), jnp.int32),
            pltpu.VMEM((N + L,), jnp.int32),
            pltpu.VMEM((N + L,), jnp.int32),
        ],
        compiler_params=pltpu.CompilerParams(needs_layout_passes=False),
    )(_masked_topk_kernel)
    return f(X)


# speedup vs baseline: 82.0147x; 1.1163x over previous
import functools

import jax
import jax.numpy as jnp
from jax import lax
from jax.experimental import pallas as pl
from jax.experimental.pallas import tpu as pltpu
from jax.experimental.pallas import tpu_sc as plsc

B = 128
N = 32768
K = 64
L = 16
SHIFT = 21
NBINS = 1 << (32 - SHIFT)
NCHUNK = NBINS // L
NVECS = N // L
U = 8


def _okey(x):
    bu = lax.bitcast_convert_type(x, jnp.uint32)
    m = jnp.where(
        bu >= jnp.uint32(0x80000000),
        jnp.uint32(0xFFFFFFFF),
        jnp.uint32(0x80000000),
    )
    return bu ^ m


def _topk_kernel(x_hbm, out_hbm, row_a, row_b, hist_v, ic_v, kidx_v, sem_in, sem_out):
    info = plsc.get_sparse_core_info()
    nc, ns = info.num_cores, info.num_subcores
    wid = lax.axis_index("s") * nc + lax.axis_index("c")
    rows_per = B // (nc * ns)
    base = wid * rows_per

    iota = jnp.arange(L, dtype=jnp.int32)
    ones = jnp.ones((L,), jnp.int32)
    zeros = jnp.zeros((L,), jnp.int32)
    fzeros = jnp.zeros((L,), jnp.float32)

    bufs = [row_a, row_b]
    pltpu.async_copy(x_hbm.at[base], row_a, sem_in).wait()
    h_out = [None, None]

    for r in range(rows_per):
        cur = bufs[r % 2]
        row = base + r

        def zero_body(g, c):
            for j in range(U):
                hist_v[pl.ds((g * U + j) * L, L)] = zeros
            return c

        lax.fori_loop(0, NCHUNK // U, zero_body, 0)

        def hist_body(g, c):
            for j in range(U):
                x = cur[pl.ds((g * U + j) * L, L)]
                binv = (_okey(x) >> SHIFT).astype(jnp.int32)
                plsc.addupdate_scatter(hist_v, [binv], ones)
            return c

        lax.fori_loop(0, NVECS // U, hist_body, 0)

        h_next = None
        if r + 1 < rows_per:
            if h_out[(r + 1) % 2] is not None:
                h_out[(r + 1) % 2].wait()
                h_out[(r + 1) % 2] = None
            h_next = pltpu.async_copy(
                x_hbm.at[base + r + 1], bufs[(r + 1) % 2], sem_in
            )

        def scan_body(gj, carry):
            total, bchunk, above = carry
            for j in range(U):
                c = (NCHUNK - 1) - (gj * U + j)
                v = hist_v[pl.ds(c * L, L)]
                csum = jnp.sum(v)
                hit = jnp.logical_and(total + csum >= K, bchunk < 0)
                bchunk = jnp.where(hit, c, bchunk)
                above = jnp.where(hit, total, above)
                total = total + csum
            return total, bchunk, above

        _, bchunk, above = lax.fori_loop(
            0,
            NCHUNK // U,
            scan_body,
            (jnp.int32(0), jnp.int32(-1), jnp.int32(0)),
        )

        v = hist_v[pl.ds(bchunk * L, L)]
        cs = plsc.cumsum(lax.rev(v, (0,)))
        ok = (above + cs) >= K
        p = jnp.sum(jnp.where(ok, ones, zeros))
        b = bchunk * L + (p - 1)
        floor_b = b.astype(jnp.uint32) << SHIFT

        def collect_body(g, ptr):
            for j in range(U // 2):
                i = g * (U // 2) + j
                x = cur[pl.ds(i * L, L)]
                eq = _okey(x) >= floor_b
                idxv = iota + i * L
                plsc.store_compressed(ic_v.at[pl.ds(ptr, L)], idxv, mask=eq)
                cnt = plsc.all_reduce_population_count(eq)[0]
                ptr = ptr + cnt
            return ptr

        cand = lax.fori_loop(0, NVECS // (U // 2), collect_body, jnp.int32(0))
        ic_v[pl.ds(cand, L)] = jnp.full((L,), -1, jnp.int32)
        nvec = (cand + L - 1) >> 4

        def gath(i):
            iv = ic_v[pl.ds(i * L, L)]
            valid = iv >= 0
            xg = plsc.load_gather(cur, [jnp.maximum(iv, 0)])
            return _okey(xg), iv, valid

        def count_ge_key(mid):
            def cb(i, accv):
                ug, _, valid = gath(i)
                m = jnp.logical_and(valid, ug >= mid)
                return accv + jnp.where(m, ones, zeros)

            return jnp.sum(lax.fori_loop(0, nvec, cb, zeros))

        def vsearch(j, lohi):
            lo, hi = lohi
            mid = lo + ((hi - lo) >> 1)
            geq = count_ge_key(mid) >= K
            return jnp.where(geq, mid, lo), jnp.where(geq, hi, mid)

        tkey, _ = lax.fori_loop(
            0, 32, vsearch, (floor_b, jnp.uint32(0xFFFFFFFF))
        )

        def cgt_body(i, accv):
            ug, _, valid = gath(i)
            m = jnp.logical_and(valid, ug > tkey)
            return accv + jnp.where(m, ones, zeros)

        cnt_gt = jnp.sum(lax.fori_loop(0, nvec, cgt_body, zeros))
        need_eq = K - cnt_gt

        def count_ge_idx(mid):
            def cb(i, accv):
                ug, iv, valid = gath(i)
                m = jnp.logical_and(
                    valid, jnp.logical_and(ug == tkey, iv >= mid)
                )
                return accv + jnp.where(m, ones, zeros)

            return jnp.sum(lax.fori_loop(0, nvec, cb, zeros))

        def isearch(j, lohi):
            lo, hi = lohi
            mid = (lo + hi) >> 1
            geq = count_ge_idx(mid) >= need_eq
            return jnp.where(geq, mid, lo), jnp.where(geq, hi, mid)

        icut, _ = lax.fori_loop(0, 15, isearch, (jnp.int32(0), jnp.int32(N)))

        def keep_body(i, ptr):
            ug, iv, valid = gath(i)
            keep = jnp.logical_and(
                valid,
                jnp.logical_or(
                    ug > tkey, jnp.logical_and(ug == tkey, iv >= icut)
                ),
            )
            plsc.store_compressed(kidx_v.at[pl.ds(ptr, L)], iv, mask=keep)
            cnt = plsc.all_reduce_population_count(keep)[0]
            return ptr + cnt

        lax.fori_loop(0, nvec, keep_body, jnp.int32(0))

        kept = []
        for j in range(K // L):
            iv = kidx_v[pl.ds(j * L, L)]
            kept.append((iv, plsc.load_gather(cur, [iv])))

        def wipe_body(g, c):
            for j in range(U):
                cur[pl.ds((g * U + j) * L, L)] = fzeros
            return c

        lax.fori_loop(0, NVECS // U, wipe_body, 0)
        for iv, xv in kept:
            plsc.store_scatter(cur, [iv], xv)

        h = pltpu.async_copy(cur, out_hbm.at[row], sem_out)
        h_out[r % 2] = h
        if h_next is not None:
            h_next.wait()

    for h in h_out:
        if h is not None:
            h.wait()


@jax.jit
def kernel(X):
    mesh = plsc.VectorSubcoreMesh(core_axis_name="c", subcore_axis_name="s")
    f = functools.partial(
        pl.kernel,
        mesh=mesh,
        out_type=jax.ShapeDtypeStruct((B, N), jnp.float32),
        scratch_types=[
            pltpu.VMEM((N,), jnp.float32),
            pltpu.VMEM((N,), jnp.float32),
            pltpu.VMEM((NBINS,), jnp.int32),
            pltpu.VMEM((N + L,), jnp.int32),
            pltpu.VMEM((K,), jnp.int32),
            pltpu.SemaphoreType.DMA,
            pltpu.SemaphoreType.DMA,
        ],
        compiler_params=pltpu.CompilerParams(needs_layout_passes=False),
    )(_topk_kernel)
    return f(X)


# speedup vs baseline: 227.5134x; 2.7741x over previous
import functools

import jax
import jax.numpy as jnp
from jax import lax
from jax.experimental import pallas as pl
from jax.experimental.pallas import tpu as pltpu
from jax.experimental.pallas import tpu_sc as plsc

B = 128
N = 32768
K = 64
L = 16
SHIFT = 21
NBINS = 1 << (32 - SHIFT)
NCHUNK = NBINS // L
NVECS = N // L


def _okey(x):
    bu = lax.bitcast_convert_type(x, jnp.uint32)
    m = jnp.where(
        bu >= jnp.uint32(0x80000000),
        jnp.uint32(0xFFFFFFFF),
        jnp.uint32(0x80000000),
    )
    return bu ^ m


def _topk_kernel(x_hbm, out_hbm, row_a, row_b, hist_v, ic_v, kidx_v, sem_in, sem_out):
    info = plsc.get_sparse_core_info()
    nc, ns = info.num_cores, info.num_subcores
    wid = lax.axis_index("s") * nc + lax.axis_index("c")
    rows_per = B // (nc * ns)
    base = wid * rows_per

    iota = jnp.arange(L, dtype=jnp.int32)
    ones = jnp.ones((L,), jnp.int32)
    zeros = jnp.zeros((L,), jnp.int32)
    fzeros = jnp.zeros((L,), jnp.float32)

    bufs = [row_a, row_b]
    pltpu.async_copy(x_hbm.at[base], row_a, sem_in).wait()
    h_out = [None, None]

    for r in range(rows_per):
        cur = bufs[r % 2]
        row = base + r

        @plsc.parallel_loop(0, NCHUNK, unroll=8)
        def _(i):
            hist_v[pl.ds(i * L, L)] = zeros

        @plsc.parallel_loop(0, NVECS, unroll=8)
        def _(i):
            x = cur[pl.ds(i * L, L)]
            binv = (_okey(x) >> SHIFT).astype(jnp.int32)
            plsc.addupdate_scatter(hist_v, [binv], ones)

        h_next = None
        if r + 1 < rows_per:
            if h_out[(r + 1) % 2] is not None:
                h_out[(r + 1) % 2].wait()
                h_out[(r + 1) % 2] = None
            h_next = pltpu.async_copy(
                x_hbm.at[base + r + 1], bufs[(r + 1) % 2], sem_in
            )

        def scan_body(j, carry):
            total, bchunk, above = carry
            c = (NCHUNK - 1) - j
            v = hist_v[pl.ds(c * L, L)]
            csum = jnp.sum(v)
            hit = jnp.logical_and(total + csum >= K, bchunk < 0)
            bchunk = jnp.where(hit, c, bchunk)
            above = jnp.where(hit, total, above)
            return total + csum, bchunk, above

        _, bchunk, above = plsc.parallel_loop(
            0,
            NCHUNK,
            unroll=8,
            carry=(jnp.int32(0), jnp.int32(-1), jnp.int32(0)),
        )(scan_body)

        v = hist_v[pl.ds(bchunk * L, L)]
        cs = plsc.cumsum(lax.rev(v, (0,)))
        ok = (above + cs) >= K
        p = jnp.sum(jnp.where(ok, ones, zeros))
        b = bchunk * L + (p - 1)
        floor_b = b.astype(jnp.uint32) << SHIFT

        def collect_body(i, ptr):
            x = cur[pl.ds(i * L, L)]
            eq = _okey(x) >= floor_b
            idxv = iota + i * L
            plsc.store_compressed(ic_v.at[pl.ds(ptr, L)], idxv, mask=eq)
            cnt = plsc.all_reduce_population_count(eq)[0]
            return ptr + cnt

        cand = plsc.parallel_loop(0, NVECS, unroll=4, carry=jnp.int32(0))(
            collect_body
        )
        ic_v[pl.ds(cand, L)] = jnp.full((L,), -1, jnp.int32)
        nvec = (cand + L - 1) >> 4

        def gath(i):
            iv = ic_v[pl.ds(i * L, L)]
            valid = iv >= 0
            xg = plsc.load_gather(cur, [jnp.maximum(iv, 0)])
            return _okey(xg), iv, valid

        def count_ge_key(mid):
            def cb(i, accv):
                ug, _, valid = gath(i)
                m = jnp.logical_and(valid, ug >= mid)
                return accv + jnp.where(m, ones, zeros)

            return jnp.sum(
                plsc.parallel_loop(0, nvec, unroll=2, carry=zeros)(cb)
            )

        def vsearch(j, lohi):
            lo, hi = lohi
            mid = lo + ((hi - lo) >> 1)
            geq = count_ge_key(mid) >= K
            return jnp.where(geq, mid, lo), jnp.where(geq, hi, mid)

        tkey, _ = lax.fori_loop(
            0, 32, vsearch, (floor_b, jnp.uint32(0xFFFFFFFF))
        )

        def cgt_body(i, accv):
            ug, _, valid = gath(i)
            m = jnp.logical_and(valid, ug > tkey)
            return accv + jnp.where(m, ones, zeros)

        cnt_gt = jnp.sum(
            plsc.parallel_loop(0, nvec, unroll=2, carry=zeros)(cgt_body)
        )
        need_eq = K - cnt_gt

        def count_ge_idx(mid):
            def cb(i, accv):
                ug, iv, valid = gath(i)
                m = jnp.logical_and(
                    valid, jnp.logical_and(ug == tkey, iv >= mid)
                )
                return accv + jnp.where(m, ones, zeros)

            return jnp.sum(
                plsc.parallel_loop(0, nvec, unroll=2, carry=zeros)(cb)
            )

        def isearch(j, lohi):
            lo, hi = lohi
            mid = (lo + hi) >> 1
            geq = count_ge_idx(mid) >= need_eq
            return jnp.where(geq, mid, lo), jnp.where(geq, hi, mid)

        icut, _ = lax.fori_loop(0, 15, isearch, (jnp.int32(0), jnp.int32(N)))

        def keep_body(i, ptr):
            ug, iv, valid = gath(i)
            keep = jnp.logical_and(
                valid,
                jnp.logical_or(
                    ug > tkey, jnp.logical_and(ug == tkey, iv >= icut)
                ),
            )
            plsc.store_compressed(kidx_v.at[pl.ds(ptr, L)], iv, mask=keep)
            cnt = plsc.all_reduce_population_count(keep)[0]
            return ptr + cnt

        plsc.parallel_loop(0, nvec, unroll=2, carry=jnp.int32(0))(keep_body)

        kept = []
        for j in range(K // L):
            iv = kidx_v[pl.ds(j * L, L)]
            kept.append((iv, plsc.load_gather(cur, [iv])))

        @plsc.parallel_loop(0, NVECS, unroll=8)
        def _(i):
            cur[pl.ds(i * L, L)] = fzeros

        for iv, xv in kept:
            plsc.store_scatter(cur, [iv], xv)

        h = pltpu.async_copy(cur, out_hbm.at[row], sem_out)
        h_out[r % 2] = h
        if h_next is not None:
            h_next.wait()

    for h in h_out:
        if h is not None:
            h.wait()


@jax.jit
def kernel(X):
    mesh = plsc.VectorSubcoreMesh(core_axis_name="c", subcore_axis_name="s")
    f = functools.partial(
        pl.kernel,
        mesh=mesh,
        out_type=jax.ShapeDtypeStruct((B, N), jnp.float32),
        scratch_types=[
            pltpu.VMEM((N,), jnp.float32),
            pltpu.VMEM((N,), jnp.float32),
            pltpu.VMEM((NBINS,), jnp.int32),
            pltpu.VMEM((N + L,), jnp.int32),
            pltpu.VMEM((K,), jnp.int32),
            pltpu.SemaphoreType.DMA,
            pltpu.SemaphoreType.DMA,
        ],
        compiler_params=pltpu.CompilerParams(needs_layout_passes=False),
    )(_topk_kernel)
    return f(X)


# speedup vs baseline: 251.6916x; 1.1063x over previous
import functools

import jax
import jax.numpy as jnp
from jax import lax
from jax.experimental import pallas as pl
from jax.experimental.pallas import tpu as pltpu
from jax.experimental.pallas import tpu_sc as plsc

B = 128
N = 32768
K = 64
L = 16
SHIFT = 21
NBINS = 1 << (32 - SHIFT)
NCHUNK = NBINS // L
NVECS = N // L


def _okey(x):
    bu = lax.bitcast_convert_type(x, jnp.uint32)
    m = jnp.where(
        bu >= jnp.uint32(0x80000000),
        jnp.uint32(0xFFFFFFFF),
        jnp.uint32(0x80000000),
    )
    return bu ^ m


def _topk_kernel(
    x_hbm, out_hbm, row_a, row_b, hist_v, ic_v, kidx_v, eqi_v, sem_in, sem_out
):
    info = plsc.get_sparse_core_info()
    nc, ns = info.num_cores, info.num_subcores
    wid = lax.axis_index("s") * nc + lax.axis_index("c")
    rows_per = B // (nc * ns)
    base = wid * rows_per

    iota = jnp.arange(L, dtype=jnp.int32)
    ones = jnp.ones((L,), jnp.int32)
    zeros = jnp.zeros((L,), jnp.int32)
    fzeros = jnp.zeros((L,), jnp.float32)

    bufs = [row_a, row_b]
    pltpu.async_copy(x_hbm.at[base], row_a, sem_in).wait()
    h_out = [None, None]

    for r in range(rows_per):
        cur = bufs[r % 2]
        row = base + r

        @plsc.parallel_loop(0, NCHUNK, unroll=8)
        def _(i):
            hist_v[pl.ds(i * L, L)] = zeros

        @plsc.parallel_loop(0, NVECS, unroll=8)
        def _(i):
            x = cur[pl.ds(i * L, L)]
            binv = (_okey(x) >> SHIFT).astype(jnp.int32)
            plsc.addupdate_scatter(hist_v, [binv], ones)

        h_next = None
        if r + 1 < rows_per:
            if h_out[(r + 1) % 2] is not None:
                h_out[(r + 1) % 2].wait()
                h_out[(r + 1) % 2] = None
            h_next = pltpu.async_copy(
                x_hbm.at[base + r + 1], bufs[(r + 1) % 2], sem_in
            )

        def scan_body(j, carry):
            total, bchunk, above = carry
            c = (NCHUNK - 1) - j
            v = hist_v[pl.ds(c * L, L)]
            csum = jnp.sum(v)
            hit = jnp.logical_and(total + csum >= K, bchunk < 0)
            bchunk = jnp.where(hit, c, bchunk)
            above = jnp.where(hit, total, above)
            return total + csum, bchunk, above

        _, bchunk, above = plsc.parallel_loop(
            0,
            NCHUNK,
            unroll=8,
            carry=(jnp.int32(0), jnp.int32(-1), jnp.int32(0)),
        )(scan_body)

        v = hist_v[pl.ds(bchunk * L, L)]
        cs = plsc.cumsum(lax.rev(v, (0,)))
        ok = (above + cs) >= K
        p = jnp.sum(jnp.where(ok, ones, zeros))
        b = bchunk * L + (p - 1)
        floor_b = b.astype(jnp.uint32) << SHIFT
        cnt_above_b = above + jnp.max(jnp.where(ok, 0, cs))

        def collect_body(i, ptr):
            x = cur[pl.ds(i * L, L)]
            eq = _okey(x) >= floor_b
            idxv = iota + i * L
            plsc.store_compressed(ic_v.at[pl.ds(ptr, L)], idxv, mask=eq)
            cnt = plsc.all_reduce_population_count(eq)[0]
            return ptr + cnt

        cand = plsc.parallel_loop(0, NVECS, unroll=4, carry=jnp.int32(0))(
            collect_body
        )
        ic_v[pl.ds(cand, L)] = jnp.full((L,), -1, jnp.int32)
        nvec = (cand + L - 1) >> 4

        def gath(i):
            iv = ic_v[pl.ds(i * L, L)]
            valid = iv >= 0
            xg = plsc.load_gather(cur, [jnp.maximum(iv, 0)])
            return _okey(xg), iv, valid

        def count_ge_key(mid):
            def cb(i, accv):
                ug, _, valid = gath(i)
                m = jnp.logical_and(valid, ug >= mid)
                return accv + jnp.where(m, ones, zeros)

            return jnp.sum(
                plsc.parallel_loop(0, nvec, unroll=2, carry=zeros)(cb)
            )

        def vsearch(j, state):
            lo, hi, cnt_hi = state
            mid = lo + ((hi - lo) >> 1)
            c = count_ge_key(mid)
            geq = c >= K
            return (
                jnp.where(geq, mid, lo),
                jnp.where(geq, hi, mid),
                jnp.where(geq, cnt_hi, c),
            )

        tkey, _, cnt_gt = lax.fori_loop(
            0,
            SHIFT,
            vsearch,
            (floor_b, floor_b + jnp.uint32(1 << SHIFT), cnt_above_b),
        )
        need_eq = K - cnt_gt

        def eq_body(i, carry):
            ptr, cnt = carry
            ug, iv, valid = gath(i)
            m = jnp.logical_and(valid, ug == tkey)
            plsc.store_compressed(eqi_v.at[pl.ds(ptr, L)], iv, mask=m)
            c = plsc.all_reduce_population_count(m)[0]
            return jnp.minimum(ptr + c, L), cnt + c

        _, cnt_eq = plsc.parallel_loop(
            0, nvec, unroll=2, carry=(jnp.int32(0), jnp.int32(0))
        )(eq_body)

        def fast_icut():
            iv16 = eqi_v[pl.ds(0, L)]
            lanemask = iota < cnt_eq
            sk, _ = plsc.sort_key_val(
                jnp.where(lanemask, iv16, jnp.int32(-1)),
                iv16,
                descending=True,
            )
            return jnp.min(
                jnp.where(iota < need_eq, sk, jnp.int32(0x7FFFFFFF))
            )

        def slow_icut():
            def count_ge_idx(mid):
                def cb(i, accv):
                    ug, iv, valid = gath(i)
                    m = jnp.logical_and(
                        valid, jnp.logical_and(ug == tkey, iv >= mid)
                    )
                    return accv + jnp.where(m, ones, zeros)

                return jnp.sum(
                    plsc.parallel_loop(0, nvec, unroll=2, carry=zeros)(cb)
                )

            def isearch(j, lohi):
                lo, hi = lohi
                mid = (lo + hi) >> 1
                geq = count_ge_idx(mid) >= need_eq
                return jnp.where(geq, mid, lo), jnp.where(geq, hi, mid)

            icut_s, _ = lax.fori_loop(
                0, 15, isearch, (jnp.int32(0), jnp.int32(N))
            )
            return icut_s

        icut = lax.cond(cnt_eq <= L, fast_icut, slow_icut)

        def keep_body(i, ptr):
            ug, iv, valid = gath(i)
            keep = jnp.logical_and(
                valid,
                jnp.logical_or(
                    ug > tkey, jnp.logical_and(ug == tkey, iv >= icut)
                ),
            )
            plsc.store_compressed(kidx_v.at[pl.ds(ptr, L)], iv, mask=keep)
            cnt = plsc.all_reduce_population_count(keep)[0]
            return ptr + cnt

        plsc.parallel_loop(0, nvec, unroll=2, carry=jnp.int32(0))(keep_body)

        kept = []
        for j in range(K // L):
            iv = kidx_v[pl.ds(j * L, L)]
            kept.append((iv, plsc.load_gather(cur, [iv])))

        @plsc.parallel_loop(0, NVECS, unroll=8)
        def _(i):
            cur[pl.ds(i * L, L)] = fzeros

        for iv, xv in kept:
            plsc.store_scatter(cur, [iv], xv)

        h = pltpu.async_copy(cur, out_hbm.at[row], sem_out)
        h_out[r % 2] = h
        if h_next is not None:
            h_next.wait()

    for h in h_out:
        if h is not None:
            h.wait()


@jax.jit
def kernel(X):
    mesh = plsc.VectorSubcoreMesh(core_axis_name="c", subcore_axis_name="s")
    f = functools.partial(
        pl.kernel,
        mesh=mesh,
        out_type=jax.ShapeDtypeStruct((B, N), jnp.float32),
        scratch_types=[
            pltpu.VMEM((N,), jnp.float32),
            pltpu.VMEM((N,), jnp.float32),
            pltpu.VMEM((NBINS,), jnp.int32),
            pltpu.VMEM((N + L,), jnp.int32),
            pltpu.VMEM((K,), jnp.int32),
            pltpu.VMEM((3 * L,), jnp.int32),
            pltpu.SemaphoreType.DMA,
            pltpu.SemaphoreType.DMA,
        ],
        compiler_params=pltpu.CompilerParams(needs_layout_passes=False),
    )(_topk_kernel)
    return f(X)
